# initial kernel scaffold (unmeasured)
import functools

import jax
import jax.numpy as jnp
from jax import lax
from jax.experimental import pallas as pl
from jax.experimental.pallas import tpu as pltpu

N_DEV = 16
N_TOK = 256
D_IN = 128
D_OUT = 256
N_EXP = 32
EXP_PER_DEV = N_EXP // N_DEV
CAP = 6
TOK_PER_DEV = N_TOK // N_DEV


def kernel(x, router_W, route_idx, expert_W):
    def body(x_ref, rw_ref, idx_ref, w_ref, out_ref,
             meta_ref, send_buf, send_sem, recv_sems):
        p = lax.axis_index("i")

        out_ref[...] = jnp.zeros_like(out_ref)

        barrier = pltpu.get_barrier_semaphore()
        for d in range(1, N_DEV):
            pl.semaphore_signal(
                barrier, inc=1,
                device_id=((p + d) % N_DEV,),
                device_id_type=pl.DeviceIdType.MESH,
            )
        pl.semaphore_wait(barrier, N_DEV - 1)

        v = idx_ref[...]
        vf = v.astype(jnp.float32)
        lane_e = lax.broadcasted_iota(jnp.float32, (N_TOK, N_EXP), 1)
        one_hot = (vf == lane_e).astype(jnp.float32)
        row_i = lax.broadcasted_iota(jnp.float32, (N_TOK, N_TOK), 0)
        col_i = lax.broadcasted_iota(jnp.float32, (N_TOK, N_TOK), 1)
        lower = (col_i <= row_i).astype(jnp.float32)
        counts = jnp.dot(lower, one_hot, preferred_element_type=jnp.float32)
        occ = jnp.sum(one_hot * counts, axis=1, keepdims=True)
        meta_ref[...] = occ

        row_iota = lax.broadcasted_iota(jnp.float32, (N_TOK, 1), 0)

        for e_local in range(EXP_PER_DEV):
            e = p * EXP_PER_DEV + e_local
            ef = e.astype(jnp.float32)
            mask_e = vf == ef
            w_bf = w_ref[e_local].astype(jnp.bfloat16)
            for c in range(CAP):
                m = jnp.logical_and(mask_e, occ == float(c + 1))
                mf = m.astype(jnp.float32)
                t = jnp.sum(row_iota * mf).astype(jnp.int32)
                exists = jnp.sum(mf) > 0.5
                xrow = x_ref[pl.ds(t, 1), :].astype(jnp.bfloat16)
                r = jnp.dot(xrow, w_bf, preferred_element_type=jnp.float32)
                dst = t // TOK_PER_DEV
                dst_row = t % TOK_PER_DEV

                @pl.when(jnp.logical_and(exists, dst == p))
                def _(r=r, dst_row=dst_row):
                    out_ref[pl.ds(dst_row, 1), :] = r

                @pl.when(jnp.logical_and(exists, dst != p))
                def _(r=r, dst=dst, dst_row=dst_row):
                    send_buf[...] = r
                    rdma = pltpu.make_async_remote_copy(
                        src_ref=send_buf,
                        dst_ref=out_ref.at[pl.ds(dst_row, 1), :],
                        send_sem=send_sem,
                        recv_sem=recv_sems.at[dst_row],
                        device_id=(dst,),
                        device_id_type=pl.DeviceIdType.MESH,
                    )
                    rdma.start()
                    rdma.wait_send()

        for j in range(TOK_PER_DEV):
            g = p * TOK_PER_DEV + j
            occ_g = pl.load(meta_ref, (pl.ds(g, 1), slice(None)))[0, 0]
            e_g = pl.load(idx_ref, (pl.ds(g, 1), slice(None)))[0, 0]
            src = e_g // EXP_PER_DEV
            expected = jnp.logical_and(occ_g <= float(CAP), src != p)

            @pl.when(expected)
            def _(j=j):
                recv = pltpu.make_async_remote_copy(
                    src_ref=send_buf,
                    dst_ref=out_ref.at[pl.ds(j, 1), :],
                    send_sem=send_sem,
                    recv_sem=recv_sems.at[j],
                    device_id=(0,),
                    device_id_type=pl.DeviceIdType.MESH,
                )
                recv.wait_recv()

        @functools.partial(pl.run_scoped, sem=pltpu.SemaphoreType.REGULAR)
        def _(sem):
            for d in range(1, N_DEV):
                pl.semaphore_signal(
                    sem, inc=1,
                    device_id=((p + d) % N_DEV,),
                    device_id_type=pl.DeviceIdType.MESH,
                )
            pl.semaphore_wait(sem, N_DEV - 1)

    return pl.pallas_call(
        body,
        out_shape=jax.ShapeDtypeStruct((TOK_PER_DEV, D_OUT), jnp.float32),
        in_specs=[pl.BlockSpec(memory_space=pltpu.VMEM)] * 4,
        out_specs=pl.BlockSpec(memory_space=pltpu.VMEM),
        scratch_shapes=[
            pltpu.VMEM((N_TOK, 1), jnp.float32),
            pltpu.VMEM((1, D_OUT), jnp.float32),
            pltpu.SemaphoreType.DMA,
            pltpu.SemaphoreType.DMA((TOK_PER_DEV,)),
        ],
        compiler_params=pltpu.CompilerParams(collective_id=0),
    )(x, router_W, route_idx, expert_W)


# baseline (device time: 23867 ns/iter reference)
import functools

import jax
import jax.numpy as jnp
from jax import lax
from jax.experimental import pallas as pl
from jax.experimental.pallas import tpu as pltpu

N_DEV = 16
N_TOK = 256
D_IN = 128
D_OUT = 256
N_EXP = 32
EXP_PER_DEV = N_EXP // N_DEV
CAP = 6
TOK_PER_DEV = N_TOK // N_DEV


def kernel(x, router_W, route_idx, expert_W):
    def body(x_ref, rw_ref, idx_ref, w_ref, out_ref,
             meta_ref, send_buf, send_sem, recv_sems):
        p = lax.axis_index("i")

        out_ref[...] = jnp.zeros_like(out_ref)

        barrier = pltpu.get_barrier_semaphore()
        for d in range(1, N_DEV):
            pl.semaphore_signal(
                barrier, inc=1,
                device_id=((p + d) % N_DEV,),
                device_id_type=pl.DeviceIdType.MESH,
            )
        pl.semaphore_wait(barrier, N_DEV - 1)

        v = idx_ref[...]
        lane_e = lax.broadcasted_iota(jnp.int32, (N_TOK, N_EXP), 1)
        one_hot = (v == lane_e).astype(jnp.float32)
        row_i = lax.broadcasted_iota(jnp.int32, (N_TOK, N_TOK), 0)
        col_i = lax.broadcasted_iota(jnp.int32, (N_TOK, N_TOK), 1)
        lower = (col_i <= row_i).astype(jnp.float32)
        counts = jnp.dot(lower, one_hot, preferred_element_type=jnp.float32)
        occ = jnp.sum(one_hot * counts, axis=1, keepdims=True)
        meta_ref[...] = occ

        row_iota = lax.broadcasted_iota(jnp.int32, (N_TOK, 1), 0).astype(jnp.float32)

        for e_local in range(EXP_PER_DEV):
            e = p * EXP_PER_DEV + e_local
            mask_e = v == e
            w_bf = w_ref[e_local].astype(jnp.bfloat16)
            for c in range(CAP):
                m = jnp.logical_and(mask_e, occ == float(c + 1))
                mf = m.astype(jnp.float32)
                t = jnp.sum(row_iota * mf).astype(jnp.int32)
                exists = jnp.sum(mf) > 0.5
                xrow = x_ref[pl.ds(t, 1), :].astype(jnp.bfloat16)
                r = jnp.dot(xrow, w_bf, preferred_element_type=jnp.float32)
                dst = t // TOK_PER_DEV
                dst_row = t % TOK_PER_DEV

                @pl.when(jnp.logical_and(exists, dst == p))
                def _(r=r, dst_row=dst_row):
                    out_ref[pl.ds(dst_row, 1), :] = r

                @pl.when(jnp.logical_and(exists, dst != p))
                def _(r=r, dst=dst, dst_row=dst_row):
                    send_buf[...] = r
                    rdma = pltpu.make_async_remote_copy(
                        src_ref=send_buf,
                        dst_ref=out_ref.at[pl.ds(dst_row, 1), :],
                        send_sem=send_sem,
                        recv_sem=recv_sems.at[dst_row],
                        device_id=(dst,),
                        device_id_type=pl.DeviceIdType.MESH,
                    )
                    rdma.start()
                    rdma.wait_send()

        for j in range(TOK_PER_DEV):
            g = p * TOK_PER_DEV + j
            occ_g = meta_ref[pl.ds(g, 1), :][0, 0]
            e_g = idx_ref[pl.ds(g, 1), :][0, 0]
            src = e_g // EXP_PER_DEV
            expected = jnp.logical_and(occ_g <= float(CAP), src != p)

            @pl.when(expected)
            def _(j=j):
                recv = pltpu.make_async_remote_copy(
                    src_ref=send_buf,
                    dst_ref=out_ref.at[pl.ds(j, 1), :],
                    send_sem=send_sem,
                    recv_sem=recv_sems.at[j],
                    device_id=(0,),
                    device_id_type=pl.DeviceIdType.MESH,
                )
                recv.wait_recv()

        @functools.partial(pl.run_scoped, sem=pltpu.SemaphoreType.REGULAR)
        def _(sem):
            for d in range(1, N_DEV):
                pl.semaphore_signal(
                    sem, inc=1,
                    device_id=((p + d) % N_DEV,),
                    device_id_type=pl.DeviceIdType.MESH,
                )
            pl.semaphore_wait(sem, N_DEV - 1)

    return pl.pallas_call(
        body,
        out_shape=jax.ShapeDtypeStruct((TOK_PER_DEV, D_OUT), jnp.float32),
        in_specs=[pl.BlockSpec(memory_space=pltpu.VMEM)] * 4,
        out_specs=pl.BlockSpec(memory_space=pltpu.VMEM),
        scratch_shapes=[
            pltpu.VMEM((N_TOK, 1), jnp.float32),
            pltpu.VMEM((1, D_OUT), jnp.float32),
            pltpu.SemaphoreType.DMA,
            pltpu.SemaphoreType.DMA((TOK_PER_DEV,)),
        ],
        compiler_params=pltpu.CompilerParams(collective_id=0),
    )(x, router_W, route_idx, expert_W)


# device time: 11330 ns/iter; 2.1065x vs baseline; 2.1065x over previous
import jax
import jax.numpy as jnp
from jax import lax
from jax.experimental import pallas as pl
from jax.experimental.pallas import tpu as pltpu

N_DEV = 16
N_TOK = 256
D_IN = 128
D_OUT = 256
N_EXP = 32
EXP_PER_DEV = N_EXP // N_DEV
CAP = 6
N_SLOT = EXP_PER_DEV * CAP
TOK_PER_DEV = N_TOK // N_DEV


def kernel(x, router_W, route_idx, expert_W):
    def body(x_ref, rw_ref, idx_ref, w_ref, out_ref,
             meta_ref, send_buf, send_sems, recv_sems):
        p = lax.axis_index("i")

        out_ref[...] = jnp.zeros_like(out_ref)

        barrier = pltpu.get_barrier_semaphore()
        for d in range(1, N_DEV):
            pl.semaphore_signal(
                barrier, inc=1,
                device_id=((p + d) % N_DEV,),
                device_id_type=pl.DeviceIdType.MESH,
            )

        v = idx_ref[...]
        lane_e = lax.broadcasted_iota(jnp.int32, (N_TOK, N_EXP), 1)
        one_hot = (v == lane_e).astype(jnp.float32)
        row_i = lax.broadcasted_iota(jnp.int32, (N_TOK, N_TOK), 0)
        col_i = lax.broadcasted_iota(jnp.int32, (N_TOK, N_TOK), 1)
        lower = (col_i <= row_i).astype(jnp.float32)
        counts = jnp.dot(lower, one_hot, preferred_element_type=jnp.float32)
        occ = jnp.sum(one_hot * counts, axis=1, keepdims=True)
        meta_ref[...] = occ

        s_iota = lax.broadcasted_iota(jnp.int32, (N_TOK, N_SLOT), 1)
        e_s = p * EXP_PER_DEV + s_iota // CAP
        c_s = (s_iota % CAP + 1).astype(jnp.float32)
        S = jnp.logical_and(v == e_s, occ == c_s).astype(jnp.float32)

        cdims = (((0,), (0,)), ((), ()))
        X = lax.dot_general(S, x_ref[...], cdims,
                            preferred_element_type=jnp.float32)
        row_iota = lax.broadcasted_iota(
            jnp.int32, (N_TOK, 1), 0).astype(jnp.float32)
        tv = lax.dot_general(S, row_iota, cdims,
                             preferred_element_type=jnp.float32)
        cnt = lax.dot_general(S, jnp.ones((N_TOK, 1), jnp.float32), cdims,
                              preferred_element_type=jnp.float32)

        X_bf = X.astype(jnp.bfloat16)
        y0 = jnp.dot(X_bf, w_ref[0].astype(jnp.bfloat16),
                     preferred_element_type=jnp.float32)
        y1 = jnp.dot(X_bf, w_ref[1].astype(jnp.bfloat16),
                     preferred_element_type=jnp.float32)
        first_half = lax.broadcasted_iota(jnp.int32, (N_SLOT, 1), 0) < CAP
        send_buf[...] = jnp.where(first_half, y0, y1)

        occ_mine = meta_ref[pl.ds(p * TOK_PER_DEV, TOK_PER_DEV), :]
        v_mine = idx_ref[pl.ds(p * TOK_PER_DEV, TOK_PER_DEV), :]
        expected = jnp.logical_and(
            occ_mine <= float(CAP), v_mine // EXP_PER_DEV != p
        ).astype(jnp.float32)

        slot_t = [tv[s, 0].astype(jnp.int32) for s in range(N_SLOT)]
        slot_on = [cnt[s, 0] > 0.5 for s in range(N_SLOT)]
        slot_dst = [t // TOK_PER_DEV for t in slot_t]
        slot_row = [t % TOK_PER_DEV for t in slot_t]

        pl.semaphore_wait(barrier, N_DEV - 1)

        def send_rdma(s):
            return pltpu.make_async_remote_copy(
                src_ref=send_buf.at[pl.ds(s, 1), :],
                dst_ref=out_ref.at[pl.ds(slot_row[s], 1), :],
                send_sem=send_sems.at[s],
                recv_sem=recv_sems.at[slot_row[s]],
                device_id=(slot_dst[s],),
                device_id_type=pl.DeviceIdType.MESH,
            )

        for s in range(N_SLOT):
            @pl.when(jnp.logical_and(slot_on[s], slot_dst[s] == p))
            def _(s=s):
                out_ref[pl.ds(slot_row[s], 1), :] = send_buf[pl.ds(s, 1), :]

            @pl.when(jnp.logical_and(slot_on[s], slot_dst[s] != p))
            def _(s=s):
                send_rdma(s).start()

        for j in range(TOK_PER_DEV):
            @pl.when(expected[j, 0] > 0.5)
            def _(j=j):
                recv = pltpu.make_async_remote_copy(
                    src_ref=send_buf.at[pl.ds(0, 1), :],
                    dst_ref=out_ref.at[pl.ds(j, 1), :],
                    send_sem=send_sems.at[0],
                    recv_sem=recv_sems.at[j],
                    device_id=(0,),
                    device_id_type=pl.DeviceIdType.MESH,
                )
                recv.wait_recv()

        for s in range(N_SLOT):
            @pl.when(jnp.logical_and(slot_on[s], slot_dst[s] != p))
            def _(s=s):
                send_rdma(s).wait_send()

    return pl.pallas_call(
        body,
        out_shape=jax.ShapeDtypeStruct((TOK_PER_DEV, D_OUT), jnp.float32),
        in_specs=[pl.BlockSpec(memory_space=pltpu.VMEM)] * 4,
        out_specs=pl.BlockSpec(memory_space=pltpu.VMEM),
        scratch_shapes=[
            pltpu.VMEM((N_TOK, 1), jnp.float32),
            pltpu.VMEM((N_SLOT, D_OUT), jnp.float32),
            pltpu.SemaphoreType.DMA((N_SLOT,)),
            pltpu.SemaphoreType.DMA((TOK_PER_DEV,)),
        ],
        compiler_params=pltpu.CompilerParams(collective_id=0),
    )(x, router_W, route_idx, expert_W)


# device time: 10763 ns/iter; 2.2175x vs baseline; 1.0527x over previous
import jax
import jax.numpy as jnp
from jax import lax
from jax.experimental import pallas as pl
from jax.experimental.pallas import tpu as pltpu

N_DEV = 16
N_TOK = 256
D_IN = 128
D_OUT = 256
N_EXP = 32
EXP_PER_DEV = N_EXP // N_DEV
CAP = 6
N_SLOT = EXP_PER_DEV * CAP
TOK_PER_DEV = N_TOK // N_DEV


def kernel(x, router_W, route_idx, expert_W):
    def body(x_ref, rw_ref, idx_ref, w_ref, out_ref,
             meta_ref, send_buf, send_sems, recv_sems):
        p = lax.axis_index("i")

        out_ref[...] = jnp.zeros_like(out_ref)

        barrier = pltpu.get_barrier_semaphore()
        for d in range(1, N_DEV):
            pl.semaphore_signal(
                barrier, inc=1,
                device_id=((p + d) % N_DEV,),
                device_id_type=pl.DeviceIdType.MESH,
            )

        v = idx_ref[...]
        lane_e = lax.broadcasted_iota(jnp.int32, (N_TOK, N_EXP), 1)
        one_hot = (v == lane_e).astype(jnp.float32)
        row_i = lax.broadcasted_iota(jnp.int32, (N_TOK, N_TOK), 0)
        col_i = lax.broadcasted_iota(jnp.int32, (N_TOK, N_TOK), 1)
        lower = (col_i <= row_i).astype(jnp.float32)
        counts = jnp.dot(lower.astype(jnp.bfloat16),
                         one_hot.astype(jnp.bfloat16),
                         preferred_element_type=jnp.float32)
        occ = jnp.sum(one_hot * counts, axis=1, keepdims=True)
        meta_ref[...] = occ

        s_iota = lax.broadcasted_iota(jnp.int32, (N_TOK, N_SLOT), 1)
        e_s = p * EXP_PER_DEV + s_iota // CAP
        c_s = (s_iota % CAP + 1).astype(jnp.float32)
        S = jnp.logical_and(v == e_s, occ == c_s).astype(jnp.float32)

        cdims = (((0,), (0,)), ((), ()))
        X = lax.dot_general(S, x_ref[...], cdims,
                            preferred_element_type=jnp.float32)
        row_iota = lax.broadcasted_iota(
            jnp.int32, (N_TOK, 2), 0).astype(jnp.float32)
        aux = jnp.where(
            lax.broadcasted_iota(jnp.int32, (N_TOK, 2), 1) == 0,
            row_iota, 1.0)
        R = lax.dot_general(S, aux, cdims,
                            preferred_element_type=jnp.float32)
        t_enc = R[:, :1] + 512.0 * (1.0 - jnp.minimum(R[:, 1:2], 1.0))

        X_bf = X.astype(jnp.bfloat16)
        y0 = jnp.dot(X_bf, w_ref[0].astype(jnp.bfloat16),
                     preferred_element_type=jnp.float32)
        y1 = jnp.dot(X_bf, w_ref[1].astype(jnp.bfloat16),
                     preferred_element_type=jnp.float32)
        first_half = lax.broadcasted_iota(jnp.int32, (N_SLOT, 1), 0) < CAP
        send_buf[...] = jnp.where(first_half, y0, y1)

        occ_mine = meta_ref[pl.ds(p * TOK_PER_DEV, TOK_PER_DEV), :]
        v_mine = idx_ref[pl.ds(p * TOK_PER_DEV, TOK_PER_DEV), :]
        expected = jnp.logical_and(
            occ_mine <= float(CAP), v_mine // EXP_PER_DEV != p
        ).astype(jnp.int32)
        j_iota = lax.broadcasted_iota(jnp.int32, (TOK_PER_DEV, 1), 0)
        exp_bits = jnp.sum(expected << j_iota)

        slot_enc = [t_enc[s, 0].astype(jnp.int32) for s in range(N_SLOT)]
        slot_on = [t < N_TOK for t in slot_enc]
        slot_dst = [t % N_TOK // TOK_PER_DEV for t in slot_enc]
        slot_row = [t % TOK_PER_DEV for t in slot_enc]

        pl.semaphore_wait(barrier, N_DEV - 1)

        def send_rdma(s):
            return pltpu.make_async_remote_copy(
                src_ref=send_buf.at[pl.ds(s, 1), :],
                dst_ref=out_ref.at[pl.ds(slot_row[s], 1), :],
                send_sem=send_sems.at[s],
                recv_sem=recv_sems.at[slot_row[s]],
                device_id=(slot_dst[s],),
                device_id_type=pl.DeviceIdType.MESH,
            )

        for s in range(N_SLOT):
            @pl.when(jnp.logical_and(slot_on[s], slot_dst[s] == p))
            def _(s=s):
                out_ref[pl.ds(slot_row[s], 1), :] = send_buf[pl.ds(s, 1), :]

            @pl.when(jnp.logical_and(slot_on[s], slot_dst[s] != p))
            def _(s=s):
                send_rdma(s).start()

        for j in range(TOK_PER_DEV):
            @pl.when(jnp.bitwise_and(lax.shift_right_logical(exp_bits, j), 1) == 1)
            def _(j=j):
                recv = pltpu.make_async_remote_copy(
                    src_ref=send_buf.at[pl.ds(0, 1), :],
                    dst_ref=out_ref.at[pl.ds(j, 1), :],
                    send_sem=send_sems.at[0],
                    recv_sem=recv_sems.at[j],
                    device_id=(0,),
                    device_id_type=pl.DeviceIdType.MESH,
                )
                recv.wait_recv()

        for s in range(N_SLOT):
            @pl.when(jnp.logical_and(slot_on[s], slot_dst[s] != p))
            def _(s=s):
                send_rdma(s).wait_send()

    return pl.pallas_call(
        body,
        out_shape=jax.ShapeDtypeStruct((TOK_PER_DEV, D_OUT), jnp.float32),
        in_specs=[pl.BlockSpec(memory_space=pltpu.VMEM)] * 4,
        out_specs=pl.BlockSpec(memory_space=pltpu.VMEM),
        scratch_shapes=[
            pltpu.VMEM((N_TOK, 1), jnp.float32),
            pltpu.VMEM((N_SLOT, D_OUT), jnp.float32),
            pltpu.SemaphoreType.DMA((N_SLOT,)),
            pltpu.SemaphoreType.DMA((TOK_PER_DEV,)),
        ],
        compiler_params=pltpu.CompilerParams(collective_id=0),
    )(x, router_W, route_idx, expert_W)


# device time: 10692 ns/iter; 2.2322x vs baseline; 1.0066x over previous
import jax
import jax.numpy as jnp
from jax import lax
from jax.experimental import pallas as pl
from jax.experimental.pallas import tpu as pltpu

N_DEV = 16
N_TOK = 256
D_IN = 128
D_OUT = 256
N_EXP = 32
EXP_PER_DEV = N_EXP // N_DEV
CAP = 6
N_SLOT = EXP_PER_DEV * CAP
TOK_PER_DEV = N_TOK // N_DEV


def kernel(x, router_W, route_idx, expert_W):
    def body(x_ref, idx_ref, w_hbm_ref, out_ref,
             meta_ref, send_buf, w_ref, send_sems, recv_sems, wcopy_sem):
        p = lax.axis_index("i")

        wcopy = pltpu.make_async_copy(w_hbm_ref, w_ref, wcopy_sem)
        wcopy.start()

        out_ref[...] = jnp.zeros_like(out_ref)

        barrier = pltpu.get_barrier_semaphore()
        for d in range(1, N_DEV):
            pl.semaphore_signal(
                barrier, inc=1,
                device_id=((p + d) % N_DEV,),
                device_id_type=pl.DeviceIdType.MESH,
            )

        v = idx_ref[...]
        lane_e = lax.broadcasted_iota(jnp.int32, (N_TOK, N_EXP), 1)
        one_hot = (v == lane_e).astype(jnp.float32)
        row_i = lax.broadcasted_iota(jnp.int32, (N_TOK, N_TOK), 0)
        col_i = lax.broadcasted_iota(jnp.int32, (N_TOK, N_TOK), 1)
        lower = (col_i <= row_i).astype(jnp.float32)
        counts = jnp.dot(lower.astype(jnp.bfloat16),
                         one_hot.astype(jnp.bfloat16),
                         preferred_element_type=jnp.float32)
        occ = jnp.sum(one_hot * counts, axis=1, keepdims=True)
        meta_ref[...] = occ

        s_iota = lax.broadcasted_iota(jnp.int32, (N_TOK, N_SLOT), 1)
        e_s = p * EXP_PER_DEV + s_iota // CAP
        c_s = (s_iota % CAP + 1).astype(jnp.float32)
        S = jnp.logical_and(v == e_s, occ == c_s).astype(jnp.float32)

        cdims = (((0,), (0,)), ((), ()))
        X = lax.dot_general(S, x_ref[...], cdims,
                            preferred_element_type=jnp.float32)
        row_iota = lax.broadcasted_iota(
            jnp.int32, (N_TOK, 2), 0).astype(jnp.float32)
        aux = jnp.where(
            lax.broadcasted_iota(jnp.int32, (N_TOK, 2), 1) == 0,
            row_iota, 1.0)
        R = lax.dot_general(S, aux, cdims,
                            preferred_element_type=jnp.float32)
        t_enc = R[:, :1] + 512.0 * (1.0 - jnp.minimum(R[:, 1:2], 1.0))

        X_bf = X.astype(jnp.bfloat16)
        wcopy.wait()
        y0 = jnp.dot(X_bf, w_ref[0].astype(jnp.bfloat16),
                     preferred_element_type=jnp.float32)
        y1 = jnp.dot(X_bf, w_ref[1].astype(jnp.bfloat16),
                     preferred_element_type=jnp.float32)
        first_half = lax.broadcasted_iota(jnp.int32, (N_SLOT, 1), 0) < CAP
        send_buf[...] = jnp.where(first_half, y0, y1)

        occ_mine = meta_ref[pl.ds(p * TOK_PER_DEV, TOK_PER_DEV), :]
        v_mine = idx_ref[pl.ds(p * TOK_PER_DEV, TOK_PER_DEV), :]
        expected = jnp.logical_and(
            occ_mine <= float(CAP), v_mine // EXP_PER_DEV != p
        ).astype(jnp.int32)
        j_iota = lax.broadcasted_iota(jnp.int32, (TOK_PER_DEV, 1), 0)
        exp_bits = jnp.sum(expected << j_iota)

        slot_enc = [t_enc[s, 0].astype(jnp.int32) for s in range(N_SLOT)]
        slot_on = [t < N_TOK for t in slot_enc]
        slot_dst = [t % N_TOK // TOK_PER_DEV for t in slot_enc]
        slot_row = [t % TOK_PER_DEV for t in slot_enc]

        pl.semaphore_wait(barrier, N_DEV - 1)

        def send_rdma(s):
            return pltpu.make_async_remote_copy(
                src_ref=send_buf.at[pl.ds(s, 1), :],
                dst_ref=out_ref.at[pl.ds(slot_row[s], 1), :],
                send_sem=send_sems.at[s],
                recv_sem=recv_sems.at[slot_row[s]],
                device_id=(slot_dst[s],),
                device_id_type=pl.DeviceIdType.MESH,
            )

        for s in range(N_SLOT):
            @pl.when(jnp.logical_and(slot_on[s], slot_dst[s] == p))
            def _(s=s):
                out_ref[pl.ds(slot_row[s], 1), :] = send_buf[pl.ds(s, 1), :]

            @pl.when(jnp.logical_and(slot_on[s], slot_dst[s] != p))
            def _(s=s):
                send_rdma(s).start()

        for j in range(TOK_PER_DEV):
            @pl.when(jnp.bitwise_and(lax.shift_right_logical(exp_bits, j), 1) == 1)
            def _(j=j):
                recv = pltpu.make_async_remote_copy(
                    src_ref=send_buf.at[pl.ds(0, 1), :],
                    dst_ref=out_ref.at[pl.ds(j, 1), :],
                    send_sem=send_sems.at[0],
                    recv_sem=recv_sems.at[j],
                    device_id=(0,),
                    device_id_type=pl.DeviceIdType.MESH,
                )
                recv.wait_recv()

        for s in range(N_SLOT):
            @pl.when(jnp.logical_and(slot_on[s], slot_dst[s] != p))
            def _(s=s):
                send_rdma(s).wait_send()

    return pl.pallas_call(
        body,
        out_shape=jax.ShapeDtypeStruct((TOK_PER_DEV, D_OUT), jnp.float32),
        in_specs=[
            pl.BlockSpec(memory_space=pltpu.VMEM),
            pl.BlockSpec(memory_space=pltpu.VMEM),
            pl.BlockSpec(memory_space=pl.ANY),
        ],
        out_specs=pl.BlockSpec(memory_space=pltpu.VMEM),
        scratch_shapes=[
            pltpu.VMEM((N_TOK, 1), jnp.float32),
            pltpu.VMEM((N_SLOT, D_OUT), jnp.float32),
            pltpu.VMEM((EXP_PER_DEV, D_IN, D_OUT), jnp.float32),
            pltpu.SemaphoreType.DMA((N_SLOT,)),
            pltpu.SemaphoreType.DMA((TOK_PER_DEV,)),
            pltpu.SemaphoreType.DMA,
        ],
        compiler_params=pltpu.CompilerParams(collective_id=0),
    )(x, route_idx, expert_W)
